# Initial kernel scaffold; baseline (speedup 1.0000x reference)
import jax
import jax.numpy as jnp
from jax.experimental import pallas as pl


def kernel(emb, pe_table):
    raise NotImplementedError("write your pallas kernel here")


# speedup vs baseline: 1.1744x; 1.1744x over previous
import jax
import jax.numpy as jnp
from jax.experimental import pallas as pl

SEQ_BLK = 512


def _add_pe_kernel(emb_ref, pe_ref, out_ref):
    d = pe_ref.shape[-1]
    pe = pe_ref[...]
    out_ref[:, :d] = emb_ref[:, :d] + pe
    out_ref[:, d:] = emb_ref[:, d:] + pe


def kernel(emb, pe_table):
    seq_len, batch, dim = emb.shape
    assert batch == 2
    emb2 = emb.reshape(seq_len, batch * dim)
    grid = (seq_len // SEQ_BLK,)
    out = pl.pallas_call(
        _add_pe_kernel,
        grid=grid,
        in_specs=[
            pl.BlockSpec((SEQ_BLK, batch * dim), lambda i: (i, 0)),
            pl.BlockSpec((SEQ_BLK, dim), lambda i: (i, 0)),
        ],
        out_specs=pl.BlockSpec((SEQ_BLK, batch * dim), lambda i: (i, 0)),
        out_shape=jax.ShapeDtypeStruct((seq_len, batch * dim), emb.dtype),
    )(emb2, pe_table)
    return out.reshape(seq_len, batch, dim)


# speedup vs baseline: 1.1760x; 1.0013x over previous
import jax
import jax.numpy as jnp
from jax.experimental import pallas as pl

SEQ_BLK = 1024


def _add_pe_kernel(emb_ref, pe_ref, out_ref):
    d = pe_ref.shape[-1]
    pe = pe_ref[...]
    out_ref[:, :d] = emb_ref[:, :d] + pe
    out_ref[:, d:] = emb_ref[:, d:] + pe


def kernel(emb, pe_table):
    seq_len, batch, dim = emb.shape
    assert batch == 2
    emb2 = emb.reshape(seq_len, batch * dim)
    grid = (seq_len // SEQ_BLK,)
    out = pl.pallas_call(
        _add_pe_kernel,
        grid=grid,
        in_specs=[
            pl.BlockSpec((SEQ_BLK, batch * dim), lambda i: (i, 0)),
            pl.BlockSpec((SEQ_BLK, dim), lambda i: (i, 0)),
        ],
        out_specs=pl.BlockSpec((SEQ_BLK, batch * dim), lambda i: (i, 0)),
        out_shape=jax.ShapeDtypeStruct((seq_len, batch * dim), emb.dtype),
    )(emb2, pe_table)
    return out.reshape(seq_len, batch, dim)
